# baseline (speedup 1.0000x reference)
import functools

import jax
import jax.numpy as jnp
import numpy as np
from jax.experimental import pallas as pl
from jax.sharding import Mesh, PartitionSpec as P

D = 1024
DFF = 4096
H = 8
BS = 64
TEMP = 0.75
SINK_ITERS = 8
B = 4
S = 4096
DH = D // H
NB = S // BS
TS = 512

_BF = jnp.bfloat16
_F32 = jnp.float32


def _ln(x, g, b):
    m = x.mean(-1, keepdims=True)
    xc = x - m
    v = (xc * xc).mean(-1, keepdims=True)
    return xc * jax.lax.rsqrt(v + 1e-5) * g + b


_ACH = 2


def _stage_a(x_ref, g_ref, b_ref, wq_ref, wk_ref, wv_ref,
             q_ref, k_ref, v_ref, qm_ref, km_ref):
    cs = TS // _ACH
    for c in range(_ACH):
        sl = slice(c * cs, (c + 1) * cs)
        xt = x_ref[sl, :]
        xn = _ln(xt, g_ref[...], b_ref[...]).astype(_BF)
        qf = jnp.dot(xn, wq_ref[...], preferred_element_type=_F32)
        kf = jnp.dot(xn, wk_ref[...], preferred_element_type=_F32)
        q_ref[sl, :] = qf.astype(_BF)
        k_ref[sl, :] = kf.astype(_BF)
        v_ref[sl, :] = jnp.dot(xn, wv_ref[...], preferred_element_type=_F32).astype(_BF)
        msl = slice(c * (cs // BS), (c + 1) * (cs // BS))
        qm_ref[msl, :] = qf.reshape(cs // BS, BS, D).mean(axis=1)
        km_ref[msl, :] = kf.reshape(cs // BS, BS, D).mean(axis=1)


def _lse(x, axis):
    m = jnp.max(x, axis=axis, keepdims=True)
    return m + jnp.log(jnp.sum(jnp.exp(x - m), axis=axis, keepdims=True))


def _stage_b2(qm_ref, km_ref, p_ref):
    bl = qm_ref.shape[0] // NB
    rs = []
    for b in range(bl):
        for h in range(H):
            qs = qm_ref[b * NB:(b + 1) * NB, h * DH:(h + 1) * DH].astype(_BF)
            ks = km_ref[b * NB:(b + 1) * NB, h * DH:(h + 1) * DH].astype(_BF)
            rs.append(jax.lax.dot_general(qs, ks, (((1,), (1,)), ((), ())),
                                          preferred_element_type=_F32))
    la = jnp.maximum(jnp.stack(rs, axis=0) * TEMP, 0.0)
    for _ in range(SINK_ITERS):
        la = la - _lse(la, -1)
        la = la - _lse(la, -2)
    p_ref[...] = jnp.exp(la).astype(_BF)


def _stage_b3(q_ref, k_ref, v_ref, p_ref, o_ref):
    q = q_ref[...]
    k = k_ref[...]
    v = v_ref[...]
    p = p_ref[0]

    kf = k.reshape(NB, BS * DH)
    vf = v.reshape(NB, BS * DH)
    kvf = jnp.concatenate([kf, vf], axis=1)
    kvs = jnp.dot(p, kvf, preferred_element_type=_F32).astype(_BF)
    ks = kvs[:, :BS * DH].reshape(NB, BS, DH)
    vs = kvs[:, BS * DH:].reshape(NB, BS, DH)
    k3 = k.reshape(NB, BS, DH)
    v3 = v.reshape(NB, BS, DH)
    kc = jnp.concatenate([k3, ks], axis=1)
    vc = jnp.concatenate([v3, vs], axis=1)
    q3 = q.reshape(NB, BS, DH)
    dots = jax.lax.dot_general(q3, kc, (((2,), (2,)), ((0,), (0,))),
                               preferred_element_type=_F32) * (DH ** -0.5)
    m = jnp.max(dots, axis=-1, keepdims=True)
    e = jnp.exp(dots - m)
    attn = (e / jnp.sum(e, axis=-1, keepdims=True)).astype(_BF)
    o = jax.lax.dot_general(attn, vc, (((2,), (1,)), ((0,), (0,))),
                            preferred_element_type=_F32)
    o_ref[...] = o.reshape(S, DH).astype(_BF)


def _stage_c(x_ref, o_ref, wp_ref, bp_ref, wo_ref, g2_ref, b2_ref,
             w1_ref, b1_ref, w2_ref, b2b_ref, out_ref):
    xt = x_ref[...]
    h1 = xt + jnp.dot(o_ref[...], wo_ref[...], preferred_element_type=_F32)
    u = jnp.dot(xt.astype(_BF), wp_ref[...], preferred_element_type=_F32) + bp_ref[...]
    hn = _ln(h1, g2_ref[...], b2_ref[...]).astype(_BF)
    t1 = jnp.dot(hn, w1_ref[...], preferred_element_type=_F32) + b1_ref[...]
    g = jax.nn.gelu(t1).astype(_BF)
    f = jnp.dot(g, w2_ref[...], preferred_element_type=_F32) + b2b_ref[...]
    out_ref[...] = u * (h1 + f)


def _pipeline(x, Wp, bp, ln1_g, ln1_b, Wq, Wk, Wv, Wo, ln2_g, ln2_b, W1, b1, W2, b2):
    bl = x.shape[0]
    xf = x.reshape(bl * S, D)
    n_tiles = (bl * S) // TS
    row_spec = pl.BlockSpec((TS, D), lambda i: (i, 0))
    full = lambda shape: pl.BlockSpec(shape, lambda i: (0,) * len(shape))
    vec = lambda: pl.BlockSpec((1, D), lambda i: (0, 0))

    wq_b, wk_b, wv_b = Wq, Wk, Wv
    mean_spec = pl.BlockSpec((TS // BS, D), lambda i: (i, 0))
    q, k, v, qm, km = pl.pallas_call(
        _stage_a,
        grid=(n_tiles,),
        in_specs=[row_spec, vec(), vec(), full((D, D)), full((D, D)), full((D, D))],
        out_specs=[row_spec, row_spec, row_spec, mean_spec, mean_spec],
        out_shape=[jax.ShapeDtypeStruct((bl * S, D), _BF)] * 3
        + [jax.ShapeDtypeStruct((bl * NB, D), _F32)] * 2,
    )(xf, ln1_g.reshape(1, D), ln1_b.reshape(1, D), wq_b, wk_b, wv_b)

    p = pl.pallas_call(
        _stage_b2,
        out_shape=jax.ShapeDtypeStruct((bl * H, NB, NB), _BF),
    )(qm, km)

    head_spec = pl.BlockSpec((S, DH), lambda b, h: (b, h))
    o = pl.pallas_call(
        _stage_b3,
        grid=(bl, H),
        in_specs=[head_spec, head_spec, head_spec,
                  pl.BlockSpec((1, NB, NB), lambda b, h: (b * H + h, 0, 0))],
        out_specs=head_spec,
        out_shape=jax.ShapeDtypeStruct((bl * S, D), _BF),
    )(q, k, v, p)

    out = pl.pallas_call(
        _stage_c,
        grid=(n_tiles,),
        in_specs=[row_spec, row_spec, full((D, D)), vec(), full((D, D)),
                  vec(), vec(), full((D, DFF)),
                  pl.BlockSpec((1, DFF), lambda i: (0, 0)),
                  full((DFF, D)), vec()],
        out_specs=row_spec,
        out_shape=jax.ShapeDtypeStruct((bl * S, D), _F32),
    )(xf, o, Wp, bp.reshape(1, D), Wo,
      ln2_g.reshape(1, D), ln2_b.reshape(1, D), W1,
      b1.reshape(1, DFF), W2, b2.reshape(1, D))

    return out.reshape(bl, S, D)


@functools.partial(jax.jit, static_argnames=())
def kernel(x, Wp, bp, ln1_g, ln1_b, Wq, Wk, Wv, Wo, ln2_g, ln2_b, W1, b1, W2, b2):
    args = (Wp.astype(_BF), bp, ln1_g, ln1_b, Wq.astype(_BF), Wk.astype(_BF),
            Wv.astype(_BF), Wo.astype(_BF), ln2_g, ln2_b, W1.astype(_BF), b1,
            W2.astype(_BF), b2)
    devs = jax.devices()
    ndev = 2 if len(devs) >= 2 else 1
    if ndev == 1:
        return _pipeline(x, *args)
    mesh = Mesh(np.array(devs[:ndev]), ("dp",))
    rep = (P(),) * 14
    f = jax.shard_map(_pipeline, mesh=mesh,
                      in_specs=(P("dp"),) + rep,
                      out_specs=P("dp"), check_vma=False)
    return f(x, *args)


# speedup vs baseline: 1.3716x; 1.3716x over previous
import functools

import jax
import jax.numpy as jnp
import numpy as np
from jax.experimental import pallas as pl
from jax.sharding import Mesh, PartitionSpec as P

D = 1024
DFF = 4096
H = 8
BS = 64
TEMP = 0.75
SINK_ITERS = 8
B = 4
S = 4096
DH = D // H
NB = S // BS
TS = 512

_BF = jnp.bfloat16
_F32 = jnp.float32


def _ln(x, g, b):
    m = x.mean(-1, keepdims=True)
    xc = x - m
    v = (xc * xc).mean(-1, keepdims=True)
    return xc * jax.lax.rsqrt(v + 1e-5) * g + b


_ACH = 2


def _stage_a(x_ref, g_ref, b_ref, wq_ref, wk_ref, wv_ref,
             q_ref, k_ref, v_ref, qm_ref, km_ref):
    cs = TS // _ACH
    for c in range(_ACH):
        sl = slice(c * cs, (c + 1) * cs)
        xt = x_ref[sl, :]
        xn = _ln(xt, g_ref[...], b_ref[...]).astype(_BF)
        qf = jnp.dot(xn, wq_ref[...], preferred_element_type=_F32)
        kf = jnp.dot(xn, wk_ref[...], preferred_element_type=_F32)
        q_ref[sl, :] = qf.astype(_BF)
        k_ref[sl, :] = kf.astype(_BF)
        v_ref[sl, :] = jnp.dot(xn, wv_ref[...], preferred_element_type=_F32).astype(_BF)
        msl = slice(c * (cs // BS), (c + 1) * (cs // BS))
        qm_ref[msl, :] = qf.reshape(cs // BS, BS, D).mean(axis=1)
        km_ref[msl, :] = kf.reshape(cs // BS, BS, D).mean(axis=1)


def _lse(x, axis):
    m = jnp.max(x, axis=axis, keepdims=True)
    return m + jnp.log(jnp.sum(jnp.exp(x - m), axis=axis, keepdims=True))


def _stage_b2(qm_ref, km_ref, p_ref):
    bl = qm_ref.shape[0] // NB
    rs = []
    for b in range(bl):
        for h in range(H):
            qs = qm_ref[b * NB:(b + 1) * NB, h * DH:(h + 1) * DH].astype(_BF)
            ks = km_ref[b * NB:(b + 1) * NB, h * DH:(h + 1) * DH].astype(_BF)
            rs.append(jax.lax.dot_general(qs, ks, (((1,), (1,)), ((), ())),
                                          preferred_element_type=_F32))
    la = jnp.maximum(jnp.stack(rs, axis=0) * TEMP, 0.0)
    for _ in range(SINK_ITERS):
        la = la - _lse(la, -1)
        la = la - _lse(la, -2)
    p_ref[...] = jnp.exp(la).astype(_BF)


def _stage_b3(q_ref, k_ref, v_ref, p_ref, o_ref):
    q = q_ref[...]
    k = k_ref[...]
    v = v_ref[...]
    p = p_ref[0]

    kf = k.reshape(NB, BS * DH)
    vf = v.reshape(NB, BS * DH)
    kvf = jnp.concatenate([kf, vf], axis=1)
    kvs = jnp.dot(p, kvf, preferred_element_type=_F32).astype(_BF)
    ks = kvs[:, :BS * DH].reshape(NB, BS, DH)
    vs = kvs[:, BS * DH:].reshape(NB, BS, DH)
    k3 = k.reshape(NB, BS, DH)
    v3 = v.reshape(NB, BS, DH)
    kc = jnp.concatenate([k3, ks], axis=1)
    vc = jnp.concatenate([v3, vs], axis=1)
    q3 = q.reshape(NB, BS, DH)
    dots = jax.lax.dot_general(q3, kc, (((2,), (2,)), ((0,), (0,))),
                               preferred_element_type=_F32) * (DH ** -0.5)
    m = jnp.max(dots, axis=-1, keepdims=True)
    e = jnp.exp(dots - m)
    attn = (e / jnp.sum(e, axis=-1, keepdims=True)).astype(_BF)
    o = jax.lax.dot_general(attn, vc, (((2,), (1,)), ((0,), (0,))),
                            preferred_element_type=_F32)
    o_ref[...] = o.reshape(S, DH).astype(_BF)


def _stage_c(x_ref, o_ref, wp_ref, bp_ref, wo_ref, g2_ref, b2_ref,
             w1_ref, b1_ref, w2_ref, b2b_ref, out_ref):
    xt = x_ref[...]
    h1 = xt + jnp.dot(o_ref[...], wo_ref[...], preferred_element_type=_F32)
    u = jnp.dot(xt.astype(_BF), wp_ref[...], preferred_element_type=_F32) + bp_ref[...]
    hn = _ln(h1, g2_ref[...], b2_ref[...]).astype(_BF)
    t1 = jnp.dot(hn, w1_ref[...], preferred_element_type=_F32) + b1_ref[...]
    g = jax.nn.gelu(t1).astype(_BF)
    f = jnp.dot(g, w2_ref[...], preferred_element_type=_F32) + b2b_ref[...]
    out_ref[...] = u * (h1 + f)


def _pipeline(x, Wp, bp, ln1_g, ln1_b, Wq, Wk, Wv, Wo, ln2_g, ln2_b, W1, b1, W2, b2):
    bl = x.shape[0]
    xf = x.reshape(bl * S, D)
    n_tiles = (bl * S) // TS
    row_spec = pl.BlockSpec((TS, D), lambda i: (i, 0))
    full = lambda shape: pl.BlockSpec(shape, lambda i: (0,) * len(shape))
    vec = lambda: pl.BlockSpec((1, D), lambda i: (0, 0))

    wq_b, wk_b, wv_b = Wq, Wk, Wv
    mean_spec = pl.BlockSpec((TS // BS, D), lambda i: (i, 0))
    q, k, v, qm, km = pl.pallas_call(
        _stage_a,
        grid=(n_tiles,),
        in_specs=[row_spec, vec(), vec(), full((D, D)), full((D, D)), full((D, D))],
        out_specs=[row_spec, row_spec, row_spec, mean_spec, mean_spec],
        out_shape=[jax.ShapeDtypeStruct((bl * S, D), _BF)] * 3
        + [jax.ShapeDtypeStruct((bl * NB, D), _F32)] * 2,
    )(xf, ln1_g.reshape(1, D), ln1_b.reshape(1, D), wq_b, wk_b, wv_b)

    p = pl.pallas_call(
        _stage_b2,
        out_shape=jax.ShapeDtypeStruct((bl * H, NB, NB), _BF),
    )(qm, km)

    head_spec = pl.BlockSpec((S, DH), lambda b, h: (b, h))
    o = pl.pallas_call(
        _stage_b3,
        grid=(bl, H),
        in_specs=[head_spec, head_spec, head_spec,
                  pl.BlockSpec((1, NB, NB), lambda b, h: (b * H + h, 0, 0))],
        out_specs=head_spec,
        out_shape=jax.ShapeDtypeStruct((bl * S, D), _BF),
    )(q, k, v, p)

    out = pl.pallas_call(
        _stage_c,
        grid=(n_tiles,),
        in_specs=[row_spec, row_spec, full((D, D)), vec(), full((D, D)),
                  vec(), vec(), full((D, DFF)),
                  pl.BlockSpec((1, DFF), lambda i: (0, 0)),
                  full((DFF, D)), vec()],
        out_specs=row_spec,
        out_shape=jax.ShapeDtypeStruct((bl * S, D), _F32),
    )(xf, o, Wp, bp.reshape(1, D), Wo,
      ln2_g.reshape(1, D), ln2_b.reshape(1, D), W1,
      b1.reshape(1, DFF), W2, b2.reshape(1, D))

    return out.reshape(bl, S, D)


@functools.partial(jax.jit, static_argnames=())
def kernel(x, Wp, bp, ln1_g, ln1_b, Wq, Wk, Wv, Wo, ln2_g, ln2_b, W1, b1, W2, b2):
    args = (Wp.astype(_BF), bp, ln1_g, ln1_b, Wq.astype(_BF), Wk.astype(_BF),
            Wv.astype(_BF), Wo.astype(_BF), ln2_g, ln2_b, W1.astype(_BF), b1,
            W2.astype(_BF), b2)
    return _pipeline(x, *args)


# speedup vs baseline: 1.3730x; 1.0010x over previous
import functools

import jax
import jax.numpy as jnp
from jax.experimental import pallas as pl

D = 1024
DFF = 4096
H = 8
BS = 64
TEMP = 0.75
SINK_ITERS = 8
B = 4
S = 4096
DH = D // H
NB = S // BS
TS = 512

_BF = jnp.bfloat16
_F32 = jnp.float32


def _ln(x, g, b):
    m = x.mean(-1, keepdims=True)
    xc = x - m
    v = (xc * xc).mean(-1, keepdims=True)
    return xc * jax.lax.rsqrt(v + 1e-5) * g + b


_ACH = 2


def _stage_a(x_ref, g_ref, b_ref, wq_ref, wk_ref, wv_ref,
             q_ref, k_ref, v_ref, qm_ref, km_ref):
    cs = TS // _ACH
    for c in range(_ACH):
        sl = slice(c * cs, (c + 1) * cs)
        xt = x_ref[sl, :]
        xn = _ln(xt, g_ref[...], b_ref[...]).astype(_BF)
        qf = jnp.dot(xn, wq_ref[...], preferred_element_type=_F32)
        kf = jnp.dot(xn, wk_ref[...], preferred_element_type=_F32)
        q_ref[sl, :] = qf.astype(_BF)
        k_ref[sl, :] = kf.astype(_BF)
        v_ref[sl, :] = jnp.dot(xn, wv_ref[...], preferred_element_type=_F32).astype(_BF)
        msl = slice(c * (cs // BS), (c + 1) * (cs // BS))
        qm_ref[msl, :] = qf.reshape(cs // BS, BS, D).mean(axis=1)
        km_ref[msl, :] = kf.reshape(cs // BS, BS, D).mean(axis=1)


def _lse(x, axis):
    m = jnp.max(x, axis=axis, keepdims=True)
    return m + jnp.log(jnp.sum(jnp.exp(x - m), axis=axis, keepdims=True))


def _stage_b2(qm_ref, km_ref, p_ref):
    bl = qm_ref.shape[0] // NB
    rs = []
    for b in range(bl):
        for h in range(H):
            qs = qm_ref[b * NB:(b + 1) * NB, h * DH:(h + 1) * DH].astype(_BF)
            ks = km_ref[b * NB:(b + 1) * NB, h * DH:(h + 1) * DH].astype(_BF)
            rs.append(jax.lax.dot_general(qs, ks, (((1,), (1,)), ((), ())),
                                          preferred_element_type=_F32))
    la = jnp.maximum(jnp.stack(rs, axis=0) * TEMP, 0.0)
    for _ in range(SINK_ITERS):
        la = la - _lse(la, -1)
        la = la - _lse(la, -2)
    p_ref[...] = jnp.exp(la).astype(_BF)


def _stage_b3(q_ref, k_ref, v_ref, p_ref, o_ref):
    q = q_ref[...]
    k = k_ref[...]
    v = v_ref[...]
    p = p_ref[0]

    kf = k.reshape(NB, BS * DH)
    vf = v.reshape(NB, BS * DH)
    kvf = jnp.concatenate([kf, vf], axis=1)
    kvs = jnp.dot(p, kvf, preferred_element_type=_F32).astype(_BF)
    ks = kvs[:, :BS * DH].reshape(NB, BS, DH)
    vs = kvs[:, BS * DH:].reshape(NB, BS, DH)
    k3 = k.reshape(NB, BS, DH)
    v3 = v.reshape(NB, BS, DH)
    kc = jnp.concatenate([k3, ks], axis=1)
    vc = jnp.concatenate([v3, vs], axis=1)
    q3 = q.reshape(NB, BS, DH)
    dots = jax.lax.dot_general(q3, kc, (((2,), (2,)), ((0,), (0,))),
                               preferred_element_type=_F32) * (DH ** -0.5)
    m = jnp.max(dots, axis=-1, keepdims=True)
    e = jnp.exp(dots - m)
    attn = (e / jnp.sum(e, axis=-1, keepdims=True)).astype(_BF)
    o = jax.lax.dot_general(attn, vc, (((2,), (1,)), ((0,), (0,))),
                            preferred_element_type=_F32)
    o_ref[...] = o.reshape(S, DH).astype(_BF)


def _stage_c(x_ref, o_ref, wp_ref, bp_ref, wo_ref, g2_ref, b2_ref,
             w1_ref, b1_ref, w2_ref, b2b_ref, out_ref):
    xt = x_ref[...]
    h1 = xt + jnp.dot(o_ref[...], wo_ref[...], preferred_element_type=_F32)
    u = jnp.dot(xt.astype(_BF), wp_ref[...], preferred_element_type=_F32) + bp_ref[...]
    hn = _ln(h1, g2_ref[...], b2_ref[...]).astype(_BF)
    t1 = jnp.dot(hn, w1_ref[...], preferred_element_type=_F32) + b1_ref[...]
    g = jax.nn.gelu(t1).astype(_BF)
    f = jnp.dot(g, w2_ref[...], preferred_element_type=_F32) + b2b_ref[...]
    out_ref[...] = u * (h1 + f)


def _pipeline(x, Wp, bp, ln1_g, ln1_b, Wq, Wk, Wv, Wo, ln2_g, ln2_b, W1, b1, W2, b2):
    bl = x.shape[0]
    xf = x.reshape(bl * S, D)
    n_tiles = (bl * S) // TS
    row_spec = pl.BlockSpec((TS, D), lambda i: (i, 0))
    full = lambda shape: pl.BlockSpec(shape, lambda i: (0,) * len(shape))
    vec = lambda: pl.BlockSpec((1, D), lambda i: (0, 0))

    wq_b, wk_b, wv_b = Wq, Wk, Wv
    mean_spec = pl.BlockSpec((TS // BS, D), lambda i: (i, 0))
    q, k, v, qm, km = pl.pallas_call(
        _stage_a,
        grid=(n_tiles,),
        in_specs=[row_spec, vec(), vec(), full((D, D)), full((D, D)), full((D, D))],
        out_specs=[row_spec, row_spec, row_spec, mean_spec, mean_spec],
        out_shape=[jax.ShapeDtypeStruct((bl * S, D), _BF)] * 3
        + [jax.ShapeDtypeStruct((bl * NB, D), _F32)] * 2,
    )(xf, ln1_g.reshape(1, D), ln1_b.reshape(1, D), wq_b, wk_b, wv_b)

    p = pl.pallas_call(
        _stage_b2,
        out_shape=jax.ShapeDtypeStruct((bl * H, NB, NB), _BF),
    )(qm, km)

    head_spec = pl.BlockSpec((S, DH), lambda b, h: (b, h))
    o = pl.pallas_call(
        _stage_b3,
        grid=(bl, H),
        in_specs=[head_spec, head_spec, head_spec,
                  pl.BlockSpec((1, NB, NB), lambda b, h: (b * H + h, 0, 0))],
        out_specs=head_spec,
        out_shape=jax.ShapeDtypeStruct((bl * S, D), _BF),
    )(q, k, v, p)

    out = pl.pallas_call(
        _stage_c,
        grid=(n_tiles,),
        in_specs=[row_spec, row_spec, full((D, D)), vec(), full((D, D)),
                  vec(), vec(), full((D, DFF)),
                  pl.BlockSpec((1, DFF), lambda i: (0, 0)),
                  full((DFF, D)), vec()],
        out_specs=row_spec,
        out_shape=jax.ShapeDtypeStruct((bl * S, D), _F32),
    )(xf, o, Wp, bp.reshape(1, D), Wo,
      ln2_g.reshape(1, D), ln2_b.reshape(1, D), W1,
      b1.reshape(1, DFF), W2, b2.reshape(1, D))

    return out.reshape(bl, S, D)


@functools.partial(jax.jit, static_argnames=())
def kernel(x, Wp, bp, ln1_g, ln1_b, Wq, Wk, Wv, Wo, ln2_g, ln2_b, W1, b1, W2, b2):
    args = (Wp.astype(_BF), bp, ln1_g, ln1_b, Wq.astype(_BF), Wk.astype(_BF),
            Wv.astype(_BF), Wo.astype(_BF), ln2_g, ln2_b, W1.astype(_BF), b1,
            W2.astype(_BF), b2)
    return _pipeline(x, *args)


# speedup vs baseline: 1.3909x; 1.0131x over previous
import functools

import jax
import jax.numpy as jnp
from jax.experimental import pallas as pl

D = 1024
DFF = 4096
H = 8
BS = 64
TEMP = 0.75
SINK_ITERS = 8
B = 4
S = 4096
DH = D // H
NB = S // BS
TS = 512
TSA = 1024

_BF = jnp.bfloat16
_F32 = jnp.float32


def _ln(x, g, b):
    m = x.mean(-1, keepdims=True)
    xc = x - m
    v = (xc * xc).mean(-1, keepdims=True)
    return xc * jax.lax.rsqrt(v + 1e-5) * g + b


_ACH = 4


def _stage_a(x_ref, g_ref, b_ref, wq_ref, wk_ref, wv_ref,
             q_ref, k_ref, v_ref, qm_ref, km_ref):
    cs = TSA // _ACH
    for c in range(_ACH):
        sl = slice(c * cs, (c + 1) * cs)
        xt = x_ref[sl, :]
        xn = _ln(xt, g_ref[...], b_ref[...]).astype(_BF)
        qf = jnp.dot(xn, wq_ref[...], preferred_element_type=_F32)
        kf = jnp.dot(xn, wk_ref[...], preferred_element_type=_F32)
        q_ref[sl, :] = qf.astype(_BF)
        k_ref[sl, :] = kf.astype(_BF)
        v_ref[sl, :] = jnp.dot(xn, wv_ref[...], preferred_element_type=_F32).astype(_BF)
        msl = slice(c * (cs // BS), (c + 1) * (cs // BS))
        qm_ref[msl, :] = qf.reshape(cs // BS, BS, D).mean(axis=1)
        km_ref[msl, :] = kf.reshape(cs // BS, BS, D).mean(axis=1)


def _lse(x, axis):
    m = jnp.max(x, axis=axis, keepdims=True)
    return m + jnp.log(jnp.sum(jnp.exp(x - m), axis=axis, keepdims=True))


def _stage_b2(qm_ref, km_ref, p_ref):
    bl = qm_ref.shape[0] // NB
    rs = []
    for b in range(bl):
        for h in range(H):
            qs = qm_ref[b * NB:(b + 1) * NB, h * DH:(h + 1) * DH].astype(_BF)
            ks = km_ref[b * NB:(b + 1) * NB, h * DH:(h + 1) * DH].astype(_BF)
            rs.append(jax.lax.dot_general(qs, ks, (((1,), (1,)), ((), ())),
                                          preferred_element_type=_F32))
    la = jnp.maximum(jnp.stack(rs, axis=0) * TEMP, 0.0)
    for _ in range(SINK_ITERS):
        la = la - _lse(la, -1)
        la = la - _lse(la, -2)
    p_ref[...] = jnp.exp(la).astype(_BF)


def _stage_b3(q_ref, k_ref, v_ref, p_ref, o_ref):
    q = q_ref[...]
    k = k_ref[...]
    v = v_ref[...]
    p = p_ref[0]

    kf = k.reshape(NB, BS * DH)
    vf = v.reshape(NB, BS * DH)
    kvf = jnp.concatenate([kf, vf], axis=1)
    kvs = jnp.dot(p, kvf, preferred_element_type=_F32).astype(_BF)
    ks = kvs[:, :BS * DH].reshape(NB, BS, DH)
    vs = kvs[:, BS * DH:].reshape(NB, BS, DH)
    k3 = k.reshape(NB, BS, DH)
    v3 = v.reshape(NB, BS, DH)
    kc = jnp.concatenate([k3, ks], axis=1)
    vc = jnp.concatenate([v3, vs], axis=1)
    q3 = q.reshape(NB, BS, DH)
    dots = jax.lax.dot_general(q3, kc, (((2,), (2,)), ((0,), (0,))),
                               preferred_element_type=_F32) * (DH ** -0.5)
    e = jnp.exp(dots)
    attn = (e / jnp.sum(e, axis=-1, keepdims=True)).astype(_BF)
    o = jax.lax.dot_general(attn, vc, (((2,), (1,)), ((0,), (0,))),
                            preferred_element_type=_F32)
    o_ref[...] = o.reshape(S, DH).astype(_BF)


def _stage_c(x_ref, o_ref, wp_ref, bp_ref, wo_ref, g2_ref, b2_ref,
             w1_ref, b1_ref, w2_ref, b2b_ref, out_ref):
    xt = x_ref[...]
    h1 = xt + jnp.dot(o_ref[...], wo_ref[...], preferred_element_type=_F32)
    u = jnp.dot(xt.astype(_BF), wp_ref[...], preferred_element_type=_F32) + bp_ref[...]
    hn = _ln(h1, g2_ref[...], b2_ref[...]).astype(_BF)
    t1 = jnp.dot(hn, w1_ref[...], preferred_element_type=_F32) + b1_ref[...]
    g = jax.nn.gelu(t1).astype(_BF)
    f = jnp.dot(g, w2_ref[...], preferred_element_type=_F32) + b2b_ref[...]
    out_ref[...] = u * (h1 + f)


def _pipeline(x, Wp, bp, ln1_g, ln1_b, Wq, Wk, Wv, Wo, ln2_g, ln2_b, W1, b1, W2, b2):
    bl = x.shape[0]
    xf = x.reshape(bl * S, D)
    n_tiles = (bl * S) // TS
    row_spec = pl.BlockSpec((TS, D), lambda i: (i, 0))
    row_spec_a = pl.BlockSpec((TSA, D), lambda i: (i, 0))
    full = lambda shape: pl.BlockSpec(shape, lambda i: (0,) * len(shape))
    vec = lambda: pl.BlockSpec((1, D), lambda i: (0, 0))

    wq_b, wk_b, wv_b = Wq, Wk, Wv
    mean_spec = pl.BlockSpec((TSA // BS, D), lambda i: (i, 0))
    q, k, v, qm, km = pl.pallas_call(
        _stage_a,
        grid=((bl * S) // TSA,),
        in_specs=[row_spec_a, vec(), vec(), full((D, D)), full((D, D)), full((D, D))],
        out_specs=[row_spec_a, row_spec_a, row_spec_a, mean_spec, mean_spec],
        out_shape=[jax.ShapeDtypeStruct((bl * S, D), _BF)] * 3
        + [jax.ShapeDtypeStruct((bl * NB, D), _F32)] * 2,
    )(xf, ln1_g.reshape(1, D), ln1_b.reshape(1, D), wq_b, wk_b, wv_b)

    p = pl.pallas_call(
        _stage_b2,
        out_shape=jax.ShapeDtypeStruct((bl * H, NB, NB), _BF),
    )(qm, km)

    head_spec = pl.BlockSpec((S, DH), lambda b, h: (b, h))
    o = pl.pallas_call(
        _stage_b3,
        grid=(bl, H),
        in_specs=[head_spec, head_spec, head_spec,
                  pl.BlockSpec((1, NB, NB), lambda b, h: (b * H + h, 0, 0))],
        out_specs=head_spec,
        out_shape=jax.ShapeDtypeStruct((bl * S, D), _BF),
    )(q, k, v, p)

    out = pl.pallas_call(
        _stage_c,
        grid=(n_tiles,),
        in_specs=[row_spec, row_spec, full((D, D)), vec(), full((D, D)),
                  vec(), vec(), full((D, DFF)),
                  pl.BlockSpec((1, DFF), lambda i: (0, 0)),
                  full((DFF, D)), vec()],
        out_specs=row_spec,
        out_shape=jax.ShapeDtypeStruct((bl * S, D), _F32),
    )(xf, o, Wp, bp.reshape(1, D), Wo,
      ln2_g.reshape(1, D), ln2_b.reshape(1, D), W1,
      b1.reshape(1, DFF), W2, b2.reshape(1, D))

    return out.reshape(bl, S, D)


@functools.partial(jax.jit, static_argnames=())
def kernel(x, Wp, bp, ln1_g, ln1_b, Wq, Wk, Wv, Wo, ln2_g, ln2_b, W1, b1, W2, b2):
    args = (Wp.astype(_BF), bp, ln1_g, ln1_b, Wq.astype(_BF), Wk.astype(_BF),
            Wv.astype(_BF), Wo.astype(_BF), ln2_g, ln2_b, W1.astype(_BF), b1,
            W2.astype(_BF), b2)
    return _pipeline(x, *args)
